# baseline (speedup 1.0000x reference)
import jax
import jax.numpy as jnp
from jax import lax
from jax.experimental import pallas as pl
from jax.experimental.pallas import tpu as pltpu
from jax.experimental.pallas import tpu_sc as plsc

EMBED_DIM = 32
MLP_HIDDEN = 64
BATCH = 16384

NC = 2
NS = 16
NW = NC * NS
BPW = BATCH // NW
K = 32
NCH = BPW // K


def _sc_gather_body(ui_hbm, ii_hbm, uemb_hbm, iemb_hbm, urows_hbm, irows_hbm,
                    idx_us, idx_is, sem_u, sem_i):
    wid = lax.axis_index("s") * NC + lax.axis_index("c")
    base = wid * BPW
    pltpu.sync_copy(ui_hbm.at[pl.ds(base, BPW)], idx_us)
    pltpu.sync_copy(ii_hbm.at[pl.ds(base, BPW)], idx_is)

    def chunk(c, carry):
        co = c * K
        waits = []
        for v in range(K // 16):
            vec_u = idx_us[pl.ds(co + v * 16, 16)]
            vec_i = idx_is[pl.ds(co + v * 16, 16)]
            for b in range(16):
                i = co + v * 16 + b
                g = base + i
                waits.append(pltpu.async_copy(
                    uemb_hbm.at[pl.ds(vec_u[b], 1)],
                    urows_hbm.at[pl.ds(g, 1)], sem_u))
                waits.append(pltpu.async_copy(
                    iemb_hbm.at[pl.ds(vec_i[b], 1)],
                    irows_hbm.at[pl.ds(g, 1)], sem_i))
        for w in waits:
            w.wait()
        return carry

    lax.fori_loop(0, NCH, chunk, 0)


def _mlp_body(u_ref, v_ref, w1t_ref, b1_ref, w2_ref, b2_ref, out_ref):
    u = u_ref[...]
    v = v_ref[...]
    h = (jnp.dot(u, w1t_ref[:EMBED_DIM, :], preferred_element_type=jnp.float32)
         + jnp.dot(v, w1t_ref[EMBED_DIM:, :], preferred_element_type=jnp.float32)
         + b1_ref[...])
    h = jnp.maximum(h, 0.0)
    o = jnp.sum(h * w2_ref[...], axis=1) + b2_ref[0, 0]
    out_ref[...] = jax.nn.sigmoid(o)


def kernel(user_indices, item_indices, user_emb, item_emb, W1, b1, W2, b2):
    mesh = plsc.VectorSubcoreMesh(core_axis_name="c", subcore_axis_name="s")
    gather = pl.kernel(
        _sc_gather_body,
        mesh=mesh,
        out_type=[
            jax.ShapeDtypeStruct((BATCH, EMBED_DIM), jnp.float32),
            jax.ShapeDtypeStruct((BATCH, EMBED_DIM), jnp.float32),
        ],
        scratch_types=[
            pltpu.VMEM((BPW,), jnp.int32),
            pltpu.VMEM((BPW,), jnp.int32),
            pltpu.SemaphoreType.DMA,
            pltpu.SemaphoreType.DMA,
        ],
    )
    u_rows, v_rows = gather(user_indices.astype(jnp.int32),
                            item_indices.astype(jnp.int32),
                            user_emb, item_emb)

    out = pl.pallas_call(
        _mlp_body,
        out_shape=jax.ShapeDtypeStruct((BATCH,), jnp.float32),
    )(u_rows, v_rows, W1.T, b1.reshape(1, MLP_HIDDEN), W2, b2.reshape(1, 1))
    return out


# speedup vs baseline: 1.1686x; 1.1686x over previous
import jax
import jax.numpy as jnp
from jax import lax
from jax.experimental import pallas as pl
from jax.experimental.pallas import tpu as pltpu
from jax.experimental.pallas import tpu_sc as plsc

NROWS = 1000000
EMBED_DIM = 32
MLP_HIDDEN = 64
BATCH = 16384

NC = 2
NS = 16
NW = NC * NS
BPW = BATCH // NW
MB = 4096


def _sc_gather_body(uq_hbm, iq_hbm, uemb_hbm, iemb_hbm, urows_hbm, irows_hbm,
                    idx_u, idx_i, rows, sem):
    wid = lax.axis_index("s") * NC + lax.axis_index("c")
    base = wid * BPW
    pltpu.sync_copy(uq_hbm.at[pl.ds(base, BPW)], idx_u)
    pltpu.sync_copy(iq_hbm.at[pl.ds(base, BPW)], idx_i)
    pltpu.async_copy(uemb_hbm.at[idx_u], rows, sem).wait()
    pltpu.sync_copy(rows, urows_hbm.at[pl.ds(base, BPW)])
    pltpu.async_copy(iemb_hbm.at[idx_i], rows, sem).wait()
    pltpu.sync_copy(rows, irows_hbm.at[pl.ds(base, BPW)])


def _mlp_body(u_ref, v_ref, us_ref, is_ref, w1t_ref, b1_ref, w2_ref, b2_ref,
              out_ref):
    usel = us_ref[...]
    isel = is_ref[...]
    u = jnp.zeros((MB, EMBED_DIM), jnp.float32)
    v = jnp.zeros((MB, EMBED_DIM), jnp.float32)
    for s in range(4):
        u = u + jnp.where(usel == s, u_ref[:, s * EMBED_DIM:(s + 1) * EMBED_DIM], 0.0)
        v = v + jnp.where(isel == s, v_ref[:, s * EMBED_DIM:(s + 1) * EMBED_DIM], 0.0)
    h = (jnp.dot(u, w1t_ref[:EMBED_DIM, :], preferred_element_type=jnp.float32)
         + jnp.dot(v, w1t_ref[EMBED_DIM:, :], preferred_element_type=jnp.float32)
         + b1_ref[...])
    h = jnp.maximum(h, 0.0)
    o = jnp.sum(h * w2_ref[...], axis=1) + b2_ref[0, 0]
    out_ref[...] = jax.nn.sigmoid(o)


def kernel(user_indices, item_indices, user_emb, item_emb, W1, b1, W2, b2):
    ui = user_indices.astype(jnp.int32)
    ii = item_indices.astype(jnp.int32)
    uembg = user_emb.reshape(NROWS // 4, 128)
    iembg = item_emb.reshape(NROWS // 4, 128)

    mesh = plsc.VectorSubcoreMesh(core_axis_name="c", subcore_axis_name="s")
    gather = pl.kernel(
        _sc_gather_body,
        mesh=mesh,
        out_type=[
            jax.ShapeDtypeStruct((BATCH, 128), jnp.float32),
            jax.ShapeDtypeStruct((BATCH, 128), jnp.float32),
        ],
        scratch_types=[
            pltpu.VMEM((BPW,), jnp.int32),
            pltpu.VMEM((BPW,), jnp.int32),
            pltpu.VMEM((BPW, 128), jnp.float32),
            pltpu.SemaphoreType.DMA,
        ],
    )
    u_rows, v_rows = gather(ui >> 2, ii >> 2, uembg, iembg)

    out = pl.pallas_call(
        _mlp_body,
        grid=(BATCH // MB,),
        in_specs=[
            pl.BlockSpec((MB, 128), lambda b: (b, 0)),
            pl.BlockSpec((MB, 128), lambda b: (b, 0)),
            pl.BlockSpec((MB, 1), lambda b: (b, 0)),
            pl.BlockSpec((MB, 1), lambda b: (b, 0)),
            pl.BlockSpec((2 * EMBED_DIM, MLP_HIDDEN), lambda b: (0, 0)),
            pl.BlockSpec((1, MLP_HIDDEN), lambda b: (0, 0)),
            pl.BlockSpec((1, MLP_HIDDEN), lambda b: (0, 0)),
            pl.BlockSpec((1, 1), lambda b: (0, 0)),
        ],
        out_specs=pl.BlockSpec((MB,), lambda b: (b,)),
        out_shape=jax.ShapeDtypeStruct((BATCH,), jnp.float32),
    )(u_rows, v_rows, (ui & 3).reshape(BATCH, 1), (ii & 3).reshape(BATCH, 1),
      W1.T, b1.reshape(1, MLP_HIDDEN), W2, b2.reshape(1, 1))
    return out
